# Initial kernel scaffold; baseline (speedup 1.0000x reference)
import jax
import jax.numpy as jnp
from jax.experimental import pallas as pl


def kernel(values, row_ids, col_ids, W, b):
    raise NotImplementedError("write your pallas kernel here")


# speedup vs baseline: 5.9596x; 5.9596x over previous
import functools

import jax
import jax.numpy as jnp
from jax import lax
from jax.experimental import pallas as pl
from jax.experimental.pallas import tpu as pltpu
from jax.experimental.pallas import tpu_sc as plsc

BATCH = 16384
INPUT_DIM = 100000
OUTPUT_DIM = 64
NNZ = 1638400

NC = 2
NS = 16
LANES = 16
NW = NC * NS
K_PER_W = NNZ // NW
CHUNK = 128
N_CHUNKS = K_PER_W // CHUNK
ROWS_PER_TILE = BATCH // NS
DJ = OUTPUT_DIM // LANES


def _sc_body(vals_hbm, rid_hbm, cid_hbm, w_hbm, out_hbm,
             cid_v, rid_v, val_v, rows_v, acc_sh, sem):
    c = lax.axis_index("c")
    s = lax.axis_index("s")
    wid = c * NS + s

    zeros16 = jnp.zeros((LANES,), jnp.float32)

    def zero_body(i, _):
        for j in range(DJ):
            rows_v[i, pl.ds(j * LANES, LANES)] = zeros16
        return 0

    lax.fori_loop(0, CHUNK, zero_body, 0)
    for t in range(ROWS_PER_TILE // CHUNK):
        pltpu.sync_copy(rows_v, acc_sh.at[pl.ds(s * ROWS_PER_TILE + t * CHUNK, CHUNK)])
    plsc.subcore_barrier()

    def chunk_body(k, _):
        base = wid * K_PER_W + k * CHUNK
        pltpu.sync_copy(cid_hbm.at[pl.ds(base, CHUNK)], cid_v)
        pltpu.sync_copy(rid_hbm.at[pl.ds(base, CHUNK)], rid_v)
        pltpu.sync_copy(vals_hbm.at[pl.ds(base, CHUNK)], val_v)
        pltpu.async_copy(w_hbm.at[cid_v], rows_v, sem).wait()

        def mul_body(g, _):
            v16 = val_v[pl.ds(g * LANES, LANES)]
            for l in range(LANES):
                i = g * LANES + l
                v = v16[l]
                for j in range(DJ):
                    sl = pl.ds(j * LANES, LANES)
                    rows_v[i, sl] = rows_v[i, sl] * v
            return 0

        lax.fori_loop(0, CHUNK // LANES, mul_body, 0)
        pltpu.sync_copy(rows_v, acc_sh.at[rid_v], add=True)
        return 0

    lax.fori_loop(0, N_CHUNKS, chunk_body, 0)
    plsc.subcore_barrier()
    pltpu.sync_copy(acc_sh.at[pl.ds(s * ROWS_PER_TILE, ROWS_PER_TILE)],
                    out_hbm.at[c, pl.ds(s * ROWS_PER_TILE, ROWS_PER_TILE)])


_sc_kernel = functools.partial(
    pl.kernel,
    out_type=jax.ShapeDtypeStruct((NC, BATCH, OUTPUT_DIM), jnp.float32),
    mesh=plsc.VectorSubcoreMesh(core_axis_name="c", subcore_axis_name="s"),
    compiler_params=pltpu.CompilerParams(use_tc_tiling_on_sc=False),
    scratch_types=[
        pltpu.VMEM((CHUNK,), jnp.int32),
        pltpu.VMEM((CHUNK,), jnp.int32),
        pltpu.VMEM((CHUNK,), jnp.float32),
        pltpu.VMEM((CHUNK, OUTPUT_DIM), jnp.float32),
        pltpu.VMEM_SHARED((BATCH, OUTPUT_DIM), jnp.float32),
        pltpu.SemaphoreType.DMA,
    ],
)(_sc_body)


def _combine_body(p_ref, b_ref, o_ref):
    o_ref[...] = p_ref[0] + p_ref[1] + b_ref[...]


def _combine(partials, b2d):
    blk = 2048
    return pl.pallas_call(
        _combine_body,
        grid=(BATCH // blk,),
        in_specs=[
            pl.BlockSpec((NC, blk, OUTPUT_DIM), lambda i: (0, i, 0)),
            pl.BlockSpec((1, OUTPUT_DIM), lambda i: (0, 0)),
        ],
        out_specs=pl.BlockSpec((blk, OUTPUT_DIM), lambda i: (i, 0)),
        out_shape=jax.ShapeDtypeStruct((BATCH, OUTPUT_DIM), jnp.float32),
    )(partials, b2d)


def kernel(values, row_ids, col_ids, W, b):
    rid = row_ids.astype(jnp.int32)
    cid = col_ids.astype(jnp.int32)
    partials = _sc_kernel(values, rid, cid, W)
    return _combine(partials, b.reshape(1, OUTPUT_DIM))


# speedup vs baseline: 25.4921x; 4.2775x over previous
import functools

import jax
import jax.numpy as jnp
from jax import lax
from jax.experimental import pallas as pl
from jax.experimental.pallas import tpu as pltpu
from jax.experimental.pallas import tpu_sc as plsc

BATCH = 16384
INPUT_DIM = 100000
OUTPUT_DIM = 64
NNZ = 1638400

NC = 2
NS = 16
LANES = 16
NW = NC * NS
K_PER_W = NNZ // NW
CHUNK = 128
SUPER = 2
NSETS = 2
SUP_NZ = SUPER * CHUNK
N_SUPER = K_PER_W // SUP_NZ
ROWS_PER_TILE = BATCH // NS
DJ = OUTPUT_DIM // LANES


def _sc_body(cid_hbm, val_hbm, rid_hbm, w_hbm, out_hbm, *refs):
    cidb = refs[0:NSETS]
    valb = refs[NSETS:2 * NSETS]
    off0 = 2 * NSETS
    ridb = [refs[off0 + r * SUPER: off0 + (r + 1) * SUPER] for r in range(NSETS)]
    off = off0 + NSETS * SUPER
    rows = [refs[off + r * SUPER: off + (r + 1) * SUPER] for r in range(NSETS)]
    off += NSETS * SUPER
    prod = refs[off: off + NSETS * SUPER]
    off += NSETS * SUPER
    rid_sc = refs[off: off + NSETS * SUPER]
    off += NSETS * SUPER
    acc_sh = refs[off]
    sem_id = refs[off + 1: off + 1 + NSETS]
    sem_g = refs[off + 1 + NSETS: off + 1 + 2 * NSETS]
    sem_sc = refs[off + 1 + 2 * NSETS: off + 1 + 2 * NSETS + NSETS * SUPER]

    c = lax.axis_index("c")
    s_ax = lax.axis_index("s")
    wid = c * NS + s_ax

    zeros16 = jnp.zeros((LANES,), jnp.float32)
    zbuf = prod[0]

    def zero_body(i, _):
        for j in range(DJ):
            zbuf[i, pl.ds(j * LANES, LANES)] = zeros16
        return 0

    lax.fori_loop(0, CHUNK, zero_body, 0)
    for t in range(ROWS_PER_TILE // CHUNK):
        pltpu.sync_copy(zbuf, acc_sh.at[pl.ds(s_ax * ROWS_PER_TILE + t * CHUNK, CHUNK)])
    plsc.subcore_barrier()

    def fetch_ids(s, r):
        gs = wid * N_SUPER + s
        pltpu.async_copy(cid_hbm.at[gs], cidb[r], sem_id[r])
        pltpu.async_copy(val_hbm.at[gs], valb[r], sem_id[r])
        for b in range(SUPER):
            pltpu.async_copy(rid_hbm.at[gs * SUPER + b], ridb[r][b], sem_id[r])

    def wait_ids(r):
        pltpu.make_async_copy(cid_hbm.at[0], cidb[r], sem_id[r]).wait()
        pltpu.make_async_copy(val_hbm.at[0], valb[r], sem_id[r]).wait()
        for b in range(SUPER):
            pltpu.make_async_copy(rid_hbm.at[0], ridb[r][b], sem_id[r]).wait()

    def issue_gathers(r):
        for b in range(SUPER):
            pltpu.async_copy(w_hbm.at[cidb[r].at[b]], rows[r][b], sem_g[r])

    def wait_gathers(r):
        for b in range(SUPER):
            pltpu.make_async_copy(w_hbm.at[cidb[r].at[b]], rows[r][b],
                                  sem_g[r]).wait()

    def consume(r, first):
        for b in range(SUPER):
            p = r * SUPER + b

            @pl.when(jnp.logical_not(first))
            def _():
                pltpu.make_async_copy(prod[p], acc_sh.at[rid_sc[p]],
                                      sem_sc[p]).wait()

            DEPTH = 4

            def mul_body(g, _, b=b, p=p):
                rv = rows[r][b]
                pr = prod[p]
                v16 = valb[r][b, pl.ds(g * LANES, LANES)]
                base = g * LANES

                def load_half(k):
                    l, h = divmod(k, 2)
                    return rv[base + l, pl.ds(h * 2 * LANES, 2 * LANES)]

                fifo = [load_half(k) for k in range(DEPTH)]
                for k in range(2 * LANES):
                    if k + DEPTH < 2 * LANES:
                        fifo.append(load_half(k + DEPTH))
                    ab = fifo[k]
                    l, h = divmod(k, 2)
                    v = v16[l]
                    pa, pb = plsc.unpack(ab, format=plsc.PackFormat.INTERLEAVED)
                    pr[base + l, pl.ds(h * 2 * LANES, LANES)] = pa * v
                    pr[base + l, pl.ds(h * 2 * LANES + LANES, LANES)] = pb * v
                return 0

            lax.fori_loop(0, CHUNK // LANES, mul_body, 0)
            for g in range(CHUNK // LANES):
                sl = pl.ds(g * LANES, LANES)
                rid_sc[p][sl] = ridb[r][b][sl]
            pltpu.async_copy(prod[p], acc_sh.at[rid_sc[p]], sem_sc[p], add=True)

    fetch_ids(0, 0)
    fetch_ids(1, 1)
    wait_ids(0)
    issue_gathers(0)

    def outer(o, _):
        for r in range(NSETS):
            s = o * NSETS + r

            @pl.when(s + 1 < N_SUPER)
            def _():
                r1 = (r + 1) % NSETS
                wait_ids(r1)
                issue_gathers(r1)

            wait_gathers(r)
            consume(r, o == 0)

            @pl.when(s + 2 < N_SUPER)
            def _():
                r2 = (r + 2) % NSETS
                fetch_ids(s + 2, r2)
        return 0

    lax.fori_loop(0, N_SUPER // NSETS, outer, 0)

    for r in range(NSETS):
        for b in range(SUPER):
            p = r * SUPER + b
            pltpu.make_async_copy(prod[p], acc_sh.at[rid_sc[p]], sem_sc[p]).wait()
    plsc.subcore_barrier()
    pltpu.sync_copy(acc_sh.at[pl.ds(s_ax * ROWS_PER_TILE, ROWS_PER_TILE)],
                    out_hbm.at[c, pl.ds(s_ax * ROWS_PER_TILE, ROWS_PER_TILE)])


_scratch = (
    [pltpu.VMEM((SUPER, CHUNK), jnp.int32) for _ in range(NSETS)]
    + [pltpu.VMEM((SUPER, CHUNK), jnp.float32) for _ in range(NSETS)]
    + [pltpu.VMEM((CHUNK,), jnp.int32) for _ in range(NSETS * SUPER)]
    + [pltpu.VMEM((CHUNK, OUTPUT_DIM), jnp.bfloat16) for _ in range(NSETS * SUPER)]
    + [pltpu.VMEM((CHUNK, OUTPUT_DIM), jnp.float32) for _ in range(NSETS * SUPER)]
    + [pltpu.VMEM((CHUNK,), jnp.int32) for _ in range(NSETS * SUPER)]
    + [pltpu.VMEM_SHARED((BATCH, OUTPUT_DIM), jnp.float32)]
    + [pltpu.SemaphoreType.DMA for _ in range(2 * NSETS + NSETS * SUPER)]
)

_sc_kernel = functools.partial(
    pl.kernel,
    out_type=jax.ShapeDtypeStruct((NC, BATCH, OUTPUT_DIM), jnp.float32),
    mesh=plsc.VectorSubcoreMesh(core_axis_name="c", subcore_axis_name="s"),
    compiler_params=pltpu.CompilerParams(use_tc_tiling_on_sc=False,
                                         needs_layout_passes=False),
    scratch_types=_scratch,
)(_sc_body)


def _combine_body(p_ref, b_ref, o_ref):
    o_ref[...] = p_ref[0] + p_ref[1] + b_ref[...]


def _combine(partials, b2d):
    blk = 2048
    return pl.pallas_call(
        _combine_body,
        grid=(BATCH // blk,),
        in_specs=[
            pl.BlockSpec((NC, blk, OUTPUT_DIM), lambda i: (0, i, 0)),
            pl.BlockSpec((1, OUTPUT_DIM), lambda i: (0, 0)),
        ],
        out_specs=pl.BlockSpec((blk, OUTPUT_DIM), lambda i: (i, 0)),
        out_shape=jax.ShapeDtypeStruct((BATCH, OUTPUT_DIM), jnp.float32),
    )(partials, b2d)


_PERM = [base + off
         for base in (0, 2 * LANES)
         for l in range(LANES)
         for off in (l, LANES + l)]


def kernel(values, row_ids, col_ids, W, b):
    rid = row_ids.astype(jnp.int32)
    cid3 = col_ids.astype(jnp.int32).reshape(-1, SUPER, CHUNK)
    val3 = values.reshape(-1, SUPER, CHUNK)
    rid2 = rid.reshape(-1, CHUNK)
    wp = W[:, jnp.array(_PERM)].astype(jnp.bfloat16)
    partials = _sc_kernel(cid3, val3, rid2, wp)
    return _combine(partials, b.reshape(1, OUTPUT_DIM))
